# Initial kernel scaffold; baseline (speedup 1.0000x reference)
import jax
import jax.numpy as jnp
from jax.experimental import pallas as pl


def kernel(z, codebook):
    raise NotImplementedError("write your pallas kernel here")


# speedup vs baseline: 1.0773x; 1.0773x over previous
import functools

import jax
import jax.numpy as jnp
from jax import lax
from jax.experimental import pallas as pl
from jax.experimental.pallas import tpu as pltpu
from jax.experimental.pallas import tpu_sc as plsc

MT = 256


def _argmin_body(zf_ref, cb_ref, idx_ref):
    zf = zf_ref[...]
    cb = cb_ref[...]
    k = cb.shape[0]
    half = k // 2
    z2 = jnp.sum(zf * zf, axis=-1, keepdims=True)
    e2 = jnp.sum(cb * cb, axis=-1)
    m = lax.dot_general(zf.astype(jnp.bfloat16), cb.astype(jnp.bfloat16),
                        (((1,), (1,)), ((), ())),
                        preferred_element_type=jnp.float32)
    d = z2 - 2.0 * m + e2[None, :]

    def half_argmin(dh, base):
        mn = jnp.min(dh, axis=1)
        lane = lax.broadcasted_iota(jnp.int32, dh.shape, 1)
        cand = jnp.where(dh == mn[:, None], lane, k)
        return mn, jnp.min(cand, axis=1) + base

    v0, i0 = half_argmin(d[:, :half], 0)
    v1, i1 = half_argmin(d[:, half:], half)
    v0b = v0.astype(jnp.bfloat16).astype(jnp.float32)
    idx_ref[0, 0, :] = jnp.where(v1 < v0b, i1, i0)


def _nearest_codeword(zf, codebook):
    m, d = zf.shape
    k = codebook.shape[0]
    nblk = m // MT
    idx3 = pl.pallas_call(
        _argmin_body,
        grid=(nblk,),
        in_specs=[
            pl.BlockSpec((MT, d), lambda i: (i, 0)),
            pl.BlockSpec((k, d), lambda i: (0, 0)),
        ],
        out_specs=pl.BlockSpec((1, 1, MT), lambda i: (i, 0, 0)),
        out_shape=jax.ShapeDtypeStruct((nblk, 1, MT), jnp.int32),
    )(zf, codebook)
    return idx3.reshape(m)


def _make_sc_gather(k, d, m):
    info = plsc.get_sparse_core_info()
    nw = info.num_cores * info.num_subcores
    rows_per_w = m // nw
    mesh = plsc.VectorSubcoreMesh(core_axis_name="c", subcore_axis_name="s")

    @functools.partial(
        pl.kernel,
        out_type=jax.ShapeDtypeStruct((m, d), jnp.float32),
        mesh=mesh,
        scratch_types=[
            pltpu.VMEM((rows_per_w,), jnp.int32),
            pltpu.VMEM((rows_per_w, d), jnp.float32),
            pltpu.SemaphoreType.DMA,
        ],
        compiler_params=pltpu.CompilerParams(use_tc_tiling_on_sc=False),
    )
    def gather_kernel(cb_hbm, idx_hbm, out_hbm, idx_v, rows_v, sem):
        wid = lax.axis_index("s") * info.num_cores + lax.axis_index("c")
        base = wid * rows_per_w
        pltpu.sync_copy(idx_hbm.at[pl.ds(base, rows_per_w)], idx_v)
        pltpu.async_copy(cb_hbm.at[idx_v], rows_v, sem).wait()
        pltpu.sync_copy(rows_v, out_hbm.at[pl.ds(base, rows_per_w)])

    return gather_kernel


def kernel(z, codebook):
    b, t, d = z.shape
    k = codebook.shape[0]
    zf = z.reshape(-1, d)
    idx = _nearest_codeword(zf, codebook)
    zq = _make_sc_gather(k, d, b * t)(codebook, idx)
    return zq.reshape(b, t, d), idx.reshape(b, t)


# speedup vs baseline: 1.1099x; 1.0302x over previous
import functools

import jax
import jax.numpy as jnp
from jax import lax
from jax.experimental import pallas as pl
from jax.experimental.pallas import tpu as pltpu
from jax.experimental.pallas import tpu_sc as plsc

MT = 256


def _argmin_body(zf_ref, cb_ref, idx_ref):
    zf = zf_ref[...]
    cb = cb_ref[...]
    k = cb.shape[0]
    half = k // 2
    z2 = jnp.sum(zf * zf, axis=-1, keepdims=True)
    e2 = jnp.sum(cb * cb, axis=-1)
    m2 = lax.dot_general((zf * -2.0).astype(jnp.bfloat16),
                         cb.astype(jnp.bfloat16),
                         (((1,), (1,)), ((), ())),
                         preferred_element_type=jnp.float32)
    d = (z2 + m2) + e2[None, :]

    def half_argmin(dh, base):
        mn = jnp.min(dh, axis=1)
        lane = lax.broadcasted_iota(jnp.int32, dh.shape, 1)
        cand = jnp.where(dh == mn[:, None], lane, k)
        return mn, jnp.min(cand, axis=1) + base

    v0, i0 = half_argmin(d[:, :half], 0)
    v1, i1 = half_argmin(d[:, half:], half)
    v0b = v0.astype(jnp.bfloat16).astype(jnp.float32)
    idx_ref[0, 0, :] = jnp.where(v1 < v0b, i1, i0)


def _nearest_codeword(zf, codebook):
    m, d = zf.shape
    k = codebook.shape[0]
    nblk = m // MT
    idx3 = pl.pallas_call(
        _argmin_body,
        grid=(nblk,),
        in_specs=[
            pl.BlockSpec((MT, d), lambda i: (i, 0)),
            pl.BlockSpec((k, d), lambda i: (0, 0)),
        ],
        out_specs=pl.BlockSpec((1, 1, MT), lambda i: (i, 0, 0)),
        out_shape=jax.ShapeDtypeStruct((nblk, 1, MT), jnp.int32),
    )(zf, codebook)
    return idx3.reshape(m)


def _make_sc_gather(k, d, m):
    info = plsc.get_sparse_core_info()
    nw = info.num_cores * info.num_subcores
    rows_per_w = m // nw
    mesh = plsc.VectorSubcoreMesh(core_axis_name="c", subcore_axis_name="s")

    @functools.partial(
        pl.kernel,
        out_type=jax.ShapeDtypeStruct((m, d), jnp.float32),
        mesh=mesh,
        scratch_types=[
            pltpu.VMEM((rows_per_w,), jnp.int32),
            pltpu.VMEM((rows_per_w, d), jnp.float32),
            pltpu.SemaphoreType.DMA,
        ],
        compiler_params=pltpu.CompilerParams(use_tc_tiling_on_sc=False),
    )
    def gather_kernel(cb_hbm, idx_hbm, out_hbm, idx_v, rows_v, sem):
        wid = lax.axis_index("s") * info.num_cores + lax.axis_index("c")
        base = wid * rows_per_w
        pltpu.sync_copy(idx_hbm.at[pl.ds(base, rows_per_w)], idx_v)
        pltpu.async_copy(cb_hbm.at[idx_v], rows_v, sem).wait()
        pltpu.sync_copy(rows_v, out_hbm.at[pl.ds(base, rows_per_w)])

    return gather_kernel


def kernel(z, codebook):
    b, t, d = z.shape
    k = codebook.shape[0]
    zf = z.reshape(-1, d)
    idx = _nearest_codeword(zf, codebook)
    zq = _make_sc_gather(k, d, b * t)(codebook, idx)
    return zq.reshape(b, t, d), idx.reshape(b, t)


# speedup vs baseline: 1.7069x; 1.5379x over previous
import functools

import jax
import jax.numpy as jnp
from jax import lax
from jax.experimental import pallas as pl
from jax.experimental.pallas import tpu as pltpu
from jax.experimental.pallas import tpu_sc as plsc

MT = 2048


def _argmin_body(zf_ref, cb_ref, cbb_ref, idx_ref):
    zf = zf_ref[...]
    cb = cb_ref[...]
    k = cb.shape[0]
    half = k // 2
    z2 = jnp.sum(zf * zf, axis=-1, keepdims=True)
    e2 = jnp.sum(cb * cb, axis=-1)
    m2 = lax.dot_general((zf * -2.0).astype(jnp.bfloat16), cbb_ref[...],
                         (((1,), (1,)), ((), ())),
                         preferred_element_type=jnp.float32)
    mt = zf.shape[0]
    sb = 2048
    lanes = 128
    nch = half // lanes
    e2rep = jnp.broadcast_to(e2[None, :], (sb, k))
    lane = lax.broadcasted_iota(jnp.int32, (sb, lanes), 1)
    idx_blocks = []
    for rb in range(mt // sb):
        rs = slice(rb * sb, (rb + 1) * sb)
        z2b = z2[rs]
        hv, hi = [], []
        for h in range(2):
            acc = jnp.full((sb, lanes), jnp.inf, jnp.float32)
            astep = jnp.zeros((sb, lanes), jnp.int32)
            for c in range(nch):
                col = h * half + c * lanes
                dc = (z2b + m2[rs, col:col + lanes]) + e2rep[:, col:col + lanes]
                take = dc < acc
                acc = jnp.where(take, dc, acc)
                astep = jnp.where(take, c, astep)
            v = jnp.min(acc, axis=1)
            kc = astep * lanes + lane + h * half
            cand = jnp.where(acc == v[:, None], kc, k)
            hv.append(v)
            hi.append(jnp.min(cand, axis=1))
        v0b = hv[0].astype(jnp.bfloat16).astype(jnp.float32)
        idx_blocks.append(jnp.where(hv[1] < v0b, hi[1], hi[0]))
    idx_ref[0, 0, :] = jnp.concatenate(idx_blocks)


def _nearest_codeword(zf, codebook):
    m, d = zf.shape
    k = codebook.shape[0]
    nblk = m // MT
    idx3 = pl.pallas_call(
        _argmin_body,
        grid=(nblk,),
        in_specs=[
            pl.BlockSpec((MT, d), lambda i: (i, 0)),
            pl.BlockSpec((k, d), lambda i: (0, 0)),
            pl.BlockSpec((k, d), lambda i: (0, 0)),
        ],
        out_specs=pl.BlockSpec((1, 1, MT), lambda i: (i, 0, 0)),
        out_shape=jax.ShapeDtypeStruct((nblk, 1, MT), jnp.int32),
    )(zf, codebook, codebook.astype(jnp.bfloat16))
    return idx3.reshape(m)


def _make_sc_gather(k, d, m):
    info = plsc.get_sparse_core_info()
    nw = info.num_cores * info.num_subcores
    rows_per_w = m // nw
    mesh = plsc.VectorSubcoreMesh(core_axis_name="c", subcore_axis_name="s")

    @functools.partial(
        pl.kernel,
        out_type=jax.ShapeDtypeStruct((m, d), jnp.float32),
        mesh=mesh,
        scratch_types=[
            pltpu.VMEM((rows_per_w,), jnp.int32),
            pltpu.VMEM((rows_per_w, d), jnp.float32),
            pltpu.SemaphoreType.DMA,
        ],
        compiler_params=pltpu.CompilerParams(use_tc_tiling_on_sc=False),
    )
    def gather_kernel(cb_hbm, idx_hbm, out_hbm, idx_v, rows_v, sem):
        wid = lax.axis_index("s") * info.num_cores + lax.axis_index("c")
        base = wid * rows_per_w
        pltpu.sync_copy(idx_hbm.at[pl.ds(base, rows_per_w)], idx_v)
        pltpu.async_copy(cb_hbm.at[idx_v], rows_v, sem).wait()
        pltpu.sync_copy(rows_v, out_hbm.at[pl.ds(base, rows_per_w)])

    return gather_kernel


def kernel(z, codebook):
    b, t, d = z.shape
    k = codebook.shape[0]
    zf = z.reshape(-1, d)
    idx = _nearest_codeword(zf, codebook)
    zq = _make_sc_gather(k, d, b * t)(codebook, idx)
    return zq.reshape(b, t, d), idx.reshape(b, t)
